# baseline (speedup 1.0000x reference)
import functools

import jax
import jax.numpy as jnp
from jax import lax
from jax.experimental import pallas as pl
from jax.experimental.pallas import tpu as pltpu
from jax.experimental.pallas import tpu_sc as plsc

VOCAB = 1000000
DIM = 32
NC = 2
NS = 16
NW = NC * NS
LANES = 16

BC = 16


def _make_gather(nb: int, nl: int):
  n_flat = nb * nl
  per_w_b = nb // NW
  n_chunks = per_w_b // BC
  chunk = BC * nl
  assert nb % (NW * BC) == 0 and n_chunks % 2 == 0

  mesh = plsc.VectorSubcoreMesh(
      core_axis_name="c", subcore_axis_name="s", num_cores=NC, num_subcores=NS
  )

  @functools.partial(
      pl.kernel,
      out_type=jax.ShapeDtypeStruct((nl, DIM, nb), jnp.float32),
      mesh=mesh,
      scratch_types=[
          [pltpu.VMEM((chunk,), jnp.int32) for _ in range(2)],
          [pltpu.VMEM((chunk, DIM), jnp.float32) for _ in range(2)],
          pltpu.VMEM((nl, DIM, BC), jnp.float32),
          [pltpu.SemaphoreType.DMA for _ in range(2)],
          pltpu.SemaphoreType.DMA,
      ],
      compiler_params=pltpu.CompilerParams(
          use_tc_tiling_on_sc=False, needs_layout_passes=False
      ),
  )
  def gather_kernel(table_hbm, idx_hbm, out_hbm, idxs, gs, t_v, sgs, sw):
    wid = lax.axis_index("s") * NC + lax.axis_index("c")
    fbase = wid * per_w_b * nl
    bbase = wid * per_w_b

    def stage_and_fire(b, k):
      pltpu.sync_copy(idx_hbm.at[pl.ds(fbase + k * chunk, chunk)], idxs[b])
      pltpu.async_copy(table_hbm.at[idxs[b]], gs[b], sgs[b])

    def wb_window(k):
      return out_hbm.at[:, :, pl.ds(bbase + k * BC, BC)]

    def transpose_chunk(b):
      def l_body(l, carry):
        row_ids = lax.iota(jnp.int32, LANES) * nl + l
        for d in range(DIM):
          col_ids = jnp.full((LANES,), d, jnp.int32)
          t_v[l, d, :] = plsc.load_gather(gs[b], [row_ids, col_ids])
        return carry

      lax.fori_loop(0, nl, l_body, 0)

    for b in range(2):
      stage_and_fire(b, b)

    def body(s, carry):
      for b in range(2):
        k = s * 2 + b

        @pl.when(k >= 1)
        def _():
          pltpu.make_async_copy(t_v, wb_window(0), sw).wait()

        pltpu.make_async_copy(table_hbm.at[idxs[b]], gs[b], sgs[b]).wait()
        transpose_chunk(b)
        pltpu.async_copy(t_v, wb_window(k), sw)

        @pl.when(k + 2 < n_chunks)
        def _():
          stage_and_fire(b, k + 2)

      return carry

    lax.fori_loop(0, n_chunks // 2, body, 0)

    pltpu.make_async_copy(t_v, wb_window(0), sw).wait()

  return gather_kernel


@jax.jit
def kernel(table, input_):
  nb, nl = input_.shape
  idx = jnp.reshape(input_ | 0, (-1,))
  out = _make_gather(nb, nl)(table, idx)
  return jnp.transpose(out, (2, 0, 1))


# speedup vs baseline: 1.3131x; 1.3131x over previous
import functools

import jax
import jax.numpy as jnp
from jax import lax
from jax.experimental import pallas as pl
from jax.experimental.pallas import tpu as pltpu
from jax.experimental.pallas import tpu_sc as plsc

VOCAB = 1000000
DIM = 32
NC = 2
NS = 16
NW = NC * NS
LANES = 16

BC = 16


def _make_gather(nb: int, nl: int):
  per_w_b = nb // NW
  n_chunks = per_w_b // BC
  chunk = BC * nl
  assert nb % (NW * BC) == 0 and n_chunks % 2 == 0

  mesh = plsc.VectorSubcoreMesh(
      core_axis_name="c", subcore_axis_name="s", num_cores=NC, num_subcores=NS
  )

  @functools.partial(
      pl.kernel,
      out_type=jax.ShapeDtypeStruct((nl, nb, DIM), jnp.float32),
      mesh=mesh,
      scratch_types=[
          [pltpu.VMEM((BC, nl), jnp.int32) for _ in range(2)],
          [pltpu.VMEM((chunk,), jnp.int32) for _ in range(2)],
          [pltpu.VMEM((chunk, DIM), jnp.float32) for _ in range(2)],
          [pltpu.SemaphoreType.DMA for _ in range(2)],
          [pltpu.SemaphoreType.DMA for _ in range(2)],
      ],
      compiler_params=pltpu.CompilerParams(
          use_tc_tiling_on_sc=False, needs_layout_passes=False
      ),
  )
  def gather_kernel(table_hbm, idx_hbm, out_hbm, idxbs, idxls, gs, sgs, sws):
    wid = lax.axis_index("s") * NC + lax.axis_index("c")
    bbase = wid * per_w_b

    def stage_and_fire(b, k):
      b0 = bbase + k * BC
      pltpu.sync_copy(idx_hbm.at[pl.ds(b0, BC), :], idxbs[b])

      def l_body(l, carry):
        rows = lax.iota(jnp.int32, LANES)
        cols = jnp.full((LANES,), 0, jnp.int32) + l
        idxls[b][pl.ds(l * BC, BC)] = plsc.load_gather(idxbs[b], [rows, cols])
        return carry

      lax.fori_loop(0, nl, l_body, 0)
      pltpu.async_copy(table_hbm.at[idxls[b]], gs[b], sgs[b])

    def fire_writebacks(b, k):
      b0 = bbase + k * BC
      for l in range(nl):
        pltpu.async_copy(
            gs[b].at[pl.ds(l * BC, BC)],
            out_hbm.at[l, pl.ds(b0, BC), :],
            sws[b],
        )

    def drain_writebacks(b):
      for l in range(nl):
        pltpu.make_async_copy(
            gs[b].at[pl.ds(l * BC, BC)], out_hbm.at[0, pl.ds(0, BC), :], sws[b]
        ).wait()

    for b in range(2):
      stage_and_fire(b, b)

    def body(s, carry):
      for b in range(2):
        k = s * 2 + b
        pltpu.make_async_copy(table_hbm.at[idxls[b]], gs[b], sgs[b]).wait()
        fire_writebacks(b, k)

        @pl.when(k + 2 < n_chunks)
        def _():
          drain_writebacks(b)
          stage_and_fire(b, k + 2)

      return carry

    lax.fori_loop(0, n_chunks // 2, body, 0)

    for b in range(2):
      drain_writebacks(b)

  return gather_kernel


@jax.jit
def kernel(table, input_):
  nb, nl = input_.shape
  out = _make_gather(nb, nl)(table, input_)
  return jnp.transpose(out, (1, 0, 2))


# speedup vs baseline: 1.3820x; 1.0525x over previous
import functools

import jax
import jax.numpy as jnp
from jax import lax
from jax.experimental import pallas as pl
from jax.experimental.pallas import tpu as pltpu
from jax.experimental.pallas import tpu_sc as plsc

VOCAB = 1000000
DIM = 32
NC = 2
NS = 16
NW = NC * NS
LANES = 16

BC = 16


def _make_gather(nb: int, nl: int):
  per_w_b = nb // NW
  n_chunks = per_w_b // BC
  chunk = BC * nl
  assert nb % (NW * BC) == 0 and n_chunks % 2 == 0

  mesh = plsc.VectorSubcoreMesh(
      core_axis_name="c", subcore_axis_name="s", num_cores=NC, num_subcores=NS
  )

  @functools.partial(
      pl.kernel,
      out_type=jax.ShapeDtypeStruct((nl, DIM, nb), jnp.float32),
      mesh=mesh,
      scratch_types=[
          [pltpu.VMEM((BC, nl), jnp.int32) for _ in range(2)],
          [pltpu.VMEM((chunk,), jnp.int32) for _ in range(2)],
          [pltpu.VMEM((chunk, DIM), jnp.float32) for _ in range(2)],
          pltpu.VMEM((nl, DIM, BC), jnp.float32),
          [pltpu.SemaphoreType.DMA for _ in range(2)],
          pltpu.SemaphoreType.DMA,
      ],
      compiler_params=pltpu.CompilerParams(
          use_tc_tiling_on_sc=False, needs_layout_passes=False
      ),
  )
  def gather_kernel(table_hbm, idx_hbm, out_hbm, idxbs, idxls, gs, t_v, sgs, sw):
    wid = lax.axis_index("s") * NC + lax.axis_index("c")
    bbase = wid * per_w_b
    lane = lax.iota(jnp.int32, LANES)

    def stage_and_fire(b, k):
      b0 = bbase + k * BC
      pltpu.sync_copy(idx_hbm.at[pl.ds(b0, BC), :], idxbs[b])

      def l_body(l, carry):
        cols = jnp.full((LANES,), 0, jnp.int32) + l
        idxls[b][pl.ds(l * BC, BC)] = plsc.load_gather(idxbs[b], [lane, cols])
        return carry

      lax.fori_loop(0, nl, l_body, 0)
      pltpu.async_copy(table_hbm.at[idxls[b]], gs[b], sgs[b])

    def wb_window(k):
      return out_hbm.at[:, :, pl.ds(bbase + k * BC, BC)]

    def transpose_chunk(b):
      def l_body(l, carry):
        lsplat = jnp.full((LANES,), 0, jnp.int32) + l
        for c in range(BC):
          csplat = jnp.full((LANES,), c, jnp.int32)
          for h in range(DIM // LANES):
            v = gs[b][l * BC + c, pl.ds(h * LANES, LANES)]
            plsc.store_scatter(t_v, [lsplat, lane + h * LANES, csplat], v)
        return carry

      lax.fori_loop(0, nl, l_body, 0)

    for b in range(2):
      stage_and_fire(b, b)

    def body(s, carry):
      for b in range(2):
        k = s * 2 + b
        pltpu.make_async_copy(table_hbm.at[idxls[b]], gs[b], sgs[b]).wait()

        @pl.when(k >= 1)
        def _():
          pltpu.make_async_copy(t_v, wb_window(0), sw).wait()

        transpose_chunk(b)
        pltpu.async_copy(t_v, wb_window(k), sw)

        @pl.when(k + 2 < n_chunks)
        def _():
          stage_and_fire(b, k + 2)

      return carry

    lax.fori_loop(0, n_chunks // 2, body, 0)

    pltpu.make_async_copy(t_v, wb_window(0), sw).wait()

  return gather_kernel


@jax.jit
def kernel(table, input_):
  nb, nl = input_.shape
  out = _make_gather(nb, nl)(table, input_)
  return jnp.transpose(out, (2, 0, 1))
